# baseline (speedup 1.0000x reference)
import jax
import jax.numpy as jnp
from jax.experimental import pallas as pl
from jax.experimental.pallas import tpu as pltpu


def _im2col_s2(xpad, oh, ow):
    Bc, Hp, Wp, C = xpad.shape
    xw = xpad.reshape(Bc, Hp // 2, 2, Wp // 2, 2, C)
    taps = []
    for i in range(4):
        for j in range(4):
            pi, i2 = i % 2, i // 2
            pj, j2 = j % 2, j // 2
            taps.append(xw[:, i2:i2 + oh, pi, j2:j2 + ow, pj, :])
    return jnp.concatenate(taps, axis=-1)


def _im2col_s1(xpad, oh, ow):
    taps = []
    for i in range(3):
        for j in range(3):
            taps.append(xpad[:, i:i + oh, j:j + ow, :])
    return jnp.concatenate(taps, axis=-1)


def _pad1(x):
    return jnp.pad(x, ((0, 0), (1, 1), (1, 1), (0, 0)))


def _mm(x, w_ref, b_ref):
    out = jnp.dot(x, w_ref[...], preferred_element_type=jnp.float32)
    return out + b_ref[...]


def _sel(q, d, m):
    v = q + d - 1
    return v % m, (v // m) + 1


def _vae_kernel(p1_ref, eps_ref, yoh_ref,
                w1_ref, b1_ref, w2_ref, b2_ref, w3_ref, b3_ref,
                wfce_ref, bfce_ref, whead_ref, bhead_ref,
                wz_ref, bz_ref, wd_ref, bd_ref,
                wd1_ref, bd1_ref, wd2_ref, bd2_ref, wd3_ref, bd3_ref,
                mu_ref, lv_ref, xr_ref):
    Bc = eps_ref.shape[0]
    zd = mu_ref.shape[1]

    h = jnp.maximum(_mm(p1_ref[...], w1_ref, b1_ref), 0.0).astype(jnp.bfloat16)
    h = h.reshape(Bc, 16, 16, 128)

    pat = _im2col_s2(_pad1(h), 8, 8).reshape(Bc * 64, 2048)
    h = jnp.maximum(_mm(pat, w2_ref, b2_ref), 0.0).astype(jnp.bfloat16)
    h = h.reshape(Bc, 8, 8, 128)

    pat = _im2col_s2(_pad1(h), 4, 4).reshape(Bc * 16, 2048)
    h = jnp.maximum(_mm(pat, w3_ref, b3_ref), 0.0).astype(jnp.bfloat16)
    feature = h.reshape(Bc, 4096)

    hE = jnp.maximum(_mm(feature, wfce_ref, bfce_ref), 0.0)
    head = _mm(hE.astype(jnp.bfloat16), whead_ref, bhead_ref)
    mu = head[:, :zd]
    logvar = head[:, zd:]
    mu_ref[...] = mu
    lv_ref[...] = logvar
    z = eps_ref[...] * jnp.exp(0.5 * logvar) + mu

    zc = jnp.concatenate([z, yoh_ref[...]], axis=1).astype(jnp.bfloat16)
    hD = jnp.maximum(_mm(zc, wz_ref, bz_ref), 0.0).astype(jnp.bfloat16)
    feat = _mm(hD, wd_ref, bd_ref).astype(jnp.bfloat16)
    d = feat.reshape(Bc, 4, 4, 256)

    pat = _im2col_s1(_pad1(d), 4, 4).reshape(Bc * 16, 2304)
    o = jnp.maximum(_mm(pat, wd1_ref, bd1_ref), 0.0).astype(jnp.bfloat16)
    o = o.reshape(Bc, 4, 4, 512)
    plane = [_pad1(o[..., p * 128:(p + 1) * 128]) for p in range(4)]

    pats = []
    for qy in range(2):
        for qx in range(2):
            taps = []
            for dy in range(3):
                sy, ry = _sel(qy, dy, 2)
                for dx in range(3):
                    sx, rx = _sel(qx, dx, 2)
                    taps.append(plane[sy * 2 + sx][:, ry:ry + 4, rx:rx + 4, :])
            pats.append(jnp.concatenate(taps, axis=-1))
    pat = jnp.stack(pats, axis=0).reshape(4 * Bc * 16, 1152)
    o = jnp.maximum(_mm(pat, wd2_ref, bd2_ref), 0.0).astype(jnp.bfloat16)
    o = o.reshape(2, 2, Bc, 4, 4, 512)

    qplane = {}
    for qy in range(2):
        for qx in range(2):
            for uy in range(2):
                for ux in range(2):
                    k = uy * 2 + ux
                    qplane[(2 * qy + uy, 2 * qx + ux)] = _pad1(
                        o[qy, qx][..., k * 128:(k + 1) * 128])

    pats = []
    for ty in range(4):
        for tx in range(4):
            taps = []
            for dy in range(3):
                sy, ry = _sel(ty, dy, 4)
                for dx in range(3):
                    sx, rx = _sel(tx, dx, 4)
                    taps.append(qplane[(sy, sx)][:, ry:ry + 4, rx:rx + 4, :])
            pats.append(jnp.concatenate(taps, axis=-1))
    pat = jnp.stack(pats, axis=0).reshape(16 * Bc * 16, 1152)
    o = jax.nn.sigmoid(_mm(pat, wd3_ref, bd3_ref))
    xr_ref[...] = o.reshape(16, Bc, 16, 4)


def kernel(convE1_wm, convE1_b, convE2_wm, convE2_b, convE3_wm, convE3_b,
           fcE_wm, fcE_b, head_wm, head_b, fromZ_wm, fromZ_b,
           fcD_wm, fcD_b, convD1_wm, convD1_b, convD2_wm, convD2_b,
           convD3_wmT, convD3_b, x, y_onehot, eps):
    B = x.shape[0]
    Bc = 32 if B % 32 == 0 else B
    nsteps = B // Bc
    grid = (1, nsteps)

    xs = jnp.pad(x[:, 0].astype(jnp.bfloat16), ((0, 0), (1, 1), (1, 1)))
    p1 = jnp.stack([xs[:, i:i + 32:2, j:j + 32:2]
                    for i in range(4) for j in range(4)], axis=-1)
    p1 = p1.reshape(B * 256, 16)

    wd2 = jnp.pad(convD2_wm.reshape(1152, 4, 64),
                  ((0, 0), (0, 0), (0, 64))).reshape(1152, 512)
    bd2 = jnp.pad(convD2_b.reshape(1, 4, 64),
                  ((0, 0), (0, 0), (0, 64))).reshape(1, 512)
    wd3 = jnp.pad(jnp.transpose(convD3_wmT).reshape(9, 64, 4),
                  ((0, 0), (0, 64), (0, 0))).reshape(1152, 4)
    bd3 = convD3_b.reshape(1, 4)

    const = lambda *shape: pl.BlockSpec(shape, lambda c, i: (0,) * len(shape))
    chunk = lambda *shape: pl.BlockSpec(
        shape, lambda c, i: (c * nsteps + i,) + (0,) * (len(shape) - 1))
    shp32 = jax.ShapeDtypeStruct((B, 32), jnp.float32)

    mu, logvar, xr = pl.pallas_call(
        _vae_kernel,
        grid=grid,
        out_shape=(shp32, shp32,
                   jax.ShapeDtypeStruct((16, B, 16, 4), jnp.float32)),
        in_specs=[
            chunk(Bc * 256, 16),
            chunk(Bc, 32),
            chunk(Bc, 10),
            const(16, 128), const(1, 128),
            const(2048, 128), const(1, 128),
            const(2048, 256), const(1, 256),
            const(4096, 1024), const(1, 1024),
            const(1024, 64), const(1, 64),
            const(42, 1024), const(1, 1024),
            const(1024, 4096), const(1, 4096),
            const(2304, 512), const(1, 512),
            const(1152, 512), const(1, 512),
            const(1152, 4), const(1, 4),
        ],
        out_specs=(
            chunk(Bc, 32),
            chunk(Bc, 32),
            pl.BlockSpec((16, Bc, 16, 4), lambda c, i: (0, c * nsteps + i, 0, 0)),
        ),
        compiler_params=pltpu.CompilerParams(
            dimension_semantics=("parallel", "arbitrary")),
    )(p1, eps.astype(jnp.float32), y_onehot.astype(jnp.float32),
      convE1_wm, convE1_b, convE2_wm, convE2_b, convE3_wm, convE3_b,
      fcE_wm, fcE_b, head_wm, head_b, fromZ_wm, fromZ_b, fcD_wm, fcD_b,
      convD1_wm, convD1_b, wd2, bd2, wd3, bd3)

    xr = xr.reshape(4, 4, B, 4, 4, 2, 2)
    xr = jnp.transpose(xr, (2, 3, 0, 5, 4, 1, 6)).reshape(B, 1, 32, 32)
    return mu, logvar, xr


# speedup vs baseline: 1.0446x; 1.0446x over previous
import jax
import jax.numpy as jnp
from jax.experimental import pallas as pl
from jax.experimental.pallas import tpu as pltpu


def _im2col_s2(xpad, oh, ow):
    Bc, Hp, Wp, C = xpad.shape
    xw = xpad.reshape(Bc, Hp // 2, 2, Wp // 2, 2, C)
    taps = []
    for i in range(4):
        for j in range(4):
            pi, i2 = i % 2, i // 2
            pj, j2 = j % 2, j // 2
            taps.append(xw[:, i2:i2 + oh, pi, j2:j2 + ow, pj, :])
    return jnp.concatenate(taps, axis=-1)


def _im2col_s1(xpad, oh, ow):
    taps = []
    for i in range(3):
        for j in range(3):
            taps.append(xpad[:, i:i + oh, j:j + ow, :])
    return jnp.concatenate(taps, axis=-1)


def _pad1(x):
    return jnp.pad(x, ((0, 0), (1, 1), (1, 1), (0, 0)))


def _mm(x, w_ref, b_ref):
    out = jnp.dot(x, w_ref[...], preferred_element_type=jnp.float32)
    return out + b_ref[...]


def _sel(q, d, m):
    v = q + d - 1
    return v % m, (v // m) + 1


def _vae_kernel(p1_ref, eps_ref, yoh_ref,
                w1_ref, b1_ref, w2_ref, b2_ref, w3_ref, b3_ref,
                wfce_ref, bfce_ref, whead_ref, bhead_ref,
                wz_ref, bz_ref, wd_ref, bd_ref,
                wd1_ref, bd1_ref, wd2_ref, bd2_ref, wd3_ref, bd3_ref,
                mu_ref, lv_ref, xr_ref):
    Bc = eps_ref.shape[0]
    zd = mu_ref.shape[1]

    h = jnp.maximum(_mm(p1_ref[...], w1_ref, b1_ref), 0.0).astype(jnp.bfloat16)
    h = h.reshape(Bc, 16, 16, 128)

    pat = _im2col_s2(_pad1(h), 8, 8).reshape(Bc * 64, 2048)
    h = jnp.maximum(_mm(pat, w2_ref, b2_ref), 0.0).astype(jnp.bfloat16)
    h = h.reshape(Bc, 8, 8, 128)

    pat = _im2col_s2(_pad1(h), 4, 4).reshape(Bc * 16, 2048)
    h = jnp.maximum(_mm(pat, w3_ref, b3_ref), 0.0).astype(jnp.bfloat16)
    feature = h.reshape(Bc, 4096)

    hE = jnp.maximum(_mm(feature, wfce_ref, bfce_ref), 0.0)
    head = _mm(hE.astype(jnp.bfloat16), whead_ref, bhead_ref)
    mu = head[:, :zd]
    logvar = head[:, zd:]
    mu_ref[...] = mu
    lv_ref[...] = logvar
    z = eps_ref[...] * jnp.exp(0.5 * logvar) + mu

    zc = jnp.concatenate([z, yoh_ref[...]], axis=1).astype(jnp.bfloat16)
    hD = jnp.maximum(_mm(zc, wz_ref, bz_ref), 0.0).astype(jnp.bfloat16)
    feat = _mm(hD, wd_ref, bd_ref).astype(jnp.bfloat16)
    d = feat.reshape(Bc, 4, 4, 256)

    pat = _im2col_s1(_pad1(d), 4, 4).reshape(Bc * 16, 2304)
    o = jnp.maximum(_mm(pat, wd1_ref, bd1_ref), 0.0).astype(jnp.bfloat16)
    o = o.reshape(Bc, 4, 4, 512)
    plane = [_pad1(o[..., p * 128:(p + 1) * 128]) for p in range(4)]

    pats = []
    for qy in range(2):
        for qx in range(2):
            taps = []
            for dy in range(3):
                sy, ry = _sel(qy, dy, 2)
                for dx in range(3):
                    sx, rx = _sel(qx, dx, 2)
                    taps.append(plane[sy * 2 + sx][:, ry:ry + 4, rx:rx + 4, :])
            pats.append(jnp.concatenate(taps, axis=-1))
    pat = jnp.stack(pats, axis=0).reshape(4 * Bc * 16, 1152)
    o = jnp.maximum(_mm(pat, wd2_ref, bd2_ref), 0.0).astype(jnp.bfloat16)
    o = o.reshape(2, 2, Bc, 4, 4, 512)

    qplane = {}
    for qy in range(2):
        for qx in range(2):
            for uy in range(2):
                for ux in range(2):
                    k = uy * 2 + ux
                    qplane[(2 * qy + uy, 2 * qx + ux)] = _pad1(
                        o[qy, qx][..., k * 128:(k + 1) * 128])

    pats = []
    for ty in range(4):
        for tx in range(4):
            taps = []
            for dy in range(3):
                sy, ry = _sel(ty, dy, 4)
                for dx in range(3):
                    sx, rx = _sel(tx, dx, 4)
                    taps.append(qplane[(sy, sx)][:, ry:ry + 4, rx:rx + 4, :])
            pats.append(jnp.concatenate(taps, axis=-1))
    pat = jnp.stack(pats, axis=0).reshape(16 * Bc * 16, 1152)
    o = jax.nn.sigmoid(_mm(pat, wd3_ref, bd3_ref))
    xr_ref[...] = o.reshape(16, Bc, 16, 4)


def kernel(convE1_wm, convE1_b, convE2_wm, convE2_b, convE3_wm, convE3_b,
           fcE_wm, fcE_b, head_wm, head_b, fromZ_wm, fromZ_b,
           fcD_wm, fcD_b, convD1_wm, convD1_b, convD2_wm, convD2_b,
           convD3_wmT, convD3_b, x, y_onehot, eps):
    B = x.shape[0]
    Bc = 16 if B % 32 == 0 else B
    nsteps = B // Bc
    grid = (1, nsteps)

    xs = jnp.pad(x[:, 0].astype(jnp.bfloat16), ((0, 0), (1, 1), (1, 1)))
    p1 = jnp.stack([xs[:, i:i + 32:2, j:j + 32:2]
                    for i in range(4) for j in range(4)], axis=-1)
    p1 = p1.reshape(B * 256, 16)

    wd2 = jnp.pad(convD2_wm.reshape(1152, 4, 64),
                  ((0, 0), (0, 0), (0, 64))).reshape(1152, 512)
    bd2 = jnp.pad(convD2_b.reshape(1, 4, 64),
                  ((0, 0), (0, 0), (0, 64))).reshape(1, 512)
    wd3 = jnp.pad(jnp.transpose(convD3_wmT).reshape(9, 64, 4),
                  ((0, 0), (0, 64), (0, 0))).reshape(1152, 4)
    bd3 = convD3_b.reshape(1, 4)

    const = lambda *shape: pl.BlockSpec(shape, lambda c, i: (0,) * len(shape))
    chunk = lambda *shape: pl.BlockSpec(
        shape, lambda c, i: (c * nsteps + i,) + (0,) * (len(shape) - 1))
    shp32 = jax.ShapeDtypeStruct((B, 32), jnp.float32)

    mu, logvar, xr = pl.pallas_call(
        _vae_kernel,
        grid=grid,
        out_shape=(shp32, shp32,
                   jax.ShapeDtypeStruct((16, B, 16, 4), jnp.float32)),
        in_specs=[
            chunk(Bc * 256, 16),
            chunk(Bc, 32),
            chunk(Bc, 10),
            const(16, 128), const(1, 128),
            const(2048, 128), const(1, 128),
            const(2048, 256), const(1, 256),
            const(4096, 1024), const(1, 1024),
            const(1024, 64), const(1, 64),
            const(42, 1024), const(1, 1024),
            const(1024, 4096), const(1, 4096),
            const(2304, 512), const(1, 512),
            const(1152, 512), const(1, 512),
            const(1152, 4), const(1, 4),
        ],
        out_specs=(
            chunk(Bc, 32),
            chunk(Bc, 32),
            pl.BlockSpec((16, Bc, 16, 4), lambda c, i: (0, c * nsteps + i, 0, 0)),
        ),
        compiler_params=pltpu.CompilerParams(
            dimension_semantics=("parallel", "arbitrary")),
    )(p1, eps.astype(jnp.float32), y_onehot.astype(jnp.float32),
      convE1_wm, convE1_b, convE2_wm, convE2_b, convE3_wm, convE3_b,
      fcE_wm, fcE_b, head_wm, head_b, fromZ_wm, fromZ_b, fcD_wm, fcD_b,
      convD1_wm, convD1_b, wd2, bd2, wd3, bd3)

    xr = xr.reshape(4, 4, B, 4, 4, 2, 2)
    xr = jnp.transpose(xr, (2, 3, 0, 5, 4, 1, 6)).reshape(B, 1, 32, 32)
    return mu, logvar, xr


# speedup vs baseline: 1.0487x; 1.0040x over previous
import jax
import jax.numpy as jnp
from jax.experimental import pallas as pl
from jax.experimental.pallas import tpu as pltpu


def _im2col_s2(xpad, oh, ow):
    Bc, Hp, Wp, C = xpad.shape
    xw = xpad.reshape(Bc, Hp // 2, 2, Wp // 2, 2, C)
    taps = []
    for i in range(4):
        for j in range(4):
            pi, i2 = i % 2, i // 2
            pj, j2 = j % 2, j // 2
            taps.append(xw[:, i2:i2 + oh, pi, j2:j2 + ow, pj, :])
    return jnp.concatenate(taps, axis=-1)


def _im2col_s1(xpad, oh, ow):
    taps = []
    for i in range(3):
        for j in range(3):
            taps.append(xpad[:, i:i + oh, j:j + ow, :])
    return jnp.concatenate(taps, axis=-1)


def _pad1(x):
    return jnp.pad(x, ((0, 0), (1, 1), (1, 1), (0, 0)))


def _mm(x, w_ref, b_ref):
    out = jnp.dot(x, w_ref[...], preferred_element_type=jnp.float32)
    return out + b_ref[...]


def _sel(q, d, m):
    v = q + d - 1
    return v % m, (v // m) + 1


def _vae_kernel(p1_ref, eps_ref, yoh_ref,
                w1_ref, b1_ref, w2_ref, b2_ref, w3_ref, b3_ref,
                wfce_ref, bfce_ref, whead_ref, bhead_ref,
                wz_ref, bz_ref, wd_ref, bd_ref,
                wd1_ref, bd1_ref, wd2_ref, bd2_ref, wd3_ref, bd3_ref,
                mu_ref, lv_ref, xr_ref):
    Bc = eps_ref.shape[0]
    zd = mu_ref.shape[1]

    ht = jnp.dot(w1_ref[...], p1_ref[...],
                 preferred_element_type=jnp.float32) + b1_ref[...]
    ht = jnp.maximum(ht, 0.0).astype(jnp.bfloat16)
    h = jnp.transpose(ht).reshape(Bc, 16, 16, 128)

    pat = _im2col_s2(_pad1(h), 8, 8).reshape(Bc * 64, 2048)
    h = jnp.maximum(_mm(pat, w2_ref, b2_ref), 0.0).astype(jnp.bfloat16)
    h = h.reshape(Bc, 8, 8, 128)

    pat = _im2col_s2(_pad1(h), 4, 4).reshape(Bc * 16, 2048)
    h = jnp.maximum(_mm(pat, w3_ref, b3_ref), 0.0).astype(jnp.bfloat16)
    feature = h.reshape(Bc, 4096)

    hE = jnp.maximum(_mm(feature, wfce_ref, bfce_ref), 0.0)
    head = _mm(hE.astype(jnp.bfloat16), whead_ref, bhead_ref)
    mu = head[:, :zd]
    logvar = head[:, zd:]
    mu_ref[...] = mu
    lv_ref[...] = logvar
    z = eps_ref[...] * jnp.exp(0.5 * logvar) + mu

    zc = jnp.concatenate([z, yoh_ref[...]], axis=1).astype(jnp.bfloat16)
    hD = jnp.maximum(_mm(zc, wz_ref, bz_ref), 0.0).astype(jnp.bfloat16)
    feat = _mm(hD, wd_ref, bd_ref).astype(jnp.bfloat16)
    d = feat.reshape(Bc, 4, 4, 256)

    pat = _im2col_s1(_pad1(d), 4, 4).reshape(Bc * 16, 2304)
    o = jnp.maximum(_mm(pat, wd1_ref, bd1_ref), 0.0).astype(jnp.bfloat16)
    o = o.reshape(Bc, 4, 4, 512)
    plane = [_pad1(o[..., p * 128:(p + 1) * 128]) for p in range(4)]

    pats = []
    for qy in range(2):
        for qx in range(2):
            taps = []
            for dy in range(3):
                sy, ry = _sel(qy, dy, 2)
                for dx in range(3):
                    sx, rx = _sel(qx, dx, 2)
                    taps.append(plane[sy * 2 + sx][:, ry:ry + 4, rx:rx + 4, :])
            pats.append(jnp.concatenate(taps, axis=-1))
    pat = jnp.stack(pats, axis=0).reshape(4 * Bc * 16, 1152)
    o = jnp.maximum(_mm(pat, wd2_ref, bd2_ref), 0.0).astype(jnp.bfloat16)
    o = o.reshape(2, 2, Bc, 4, 4, 512)

    qplane = {}
    for qy in range(2):
        for qx in range(2):
            for uy in range(2):
                for ux in range(2):
                    k = uy * 2 + ux
                    qplane[(2 * qy + uy, 2 * qx + ux)] = _pad1(
                        o[qy, qx][..., k * 128:(k + 1) * 128])

    pats = []
    for ty in range(4):
        for tx in range(4):
            taps = []
            for dy in range(3):
                sy, ry = _sel(ty, dy, 4)
                for dx in range(3):
                    sx, rx = _sel(tx, dx, 4)
                    taps.append(qplane[(sy, sx)][:, ry:ry + 4, rx:rx + 4, :])
            pats.append(jnp.concatenate(taps, axis=-1))
    pat = jnp.stack(pats, axis=0).reshape(16 * Bc * 16, 1152)
    o = jax.nn.sigmoid(_mm(pat, wd3_ref, bd3_ref))
    xr_ref[...] = o.reshape(16, Bc, 16, 4)


def kernel(convE1_wm, convE1_b, convE2_wm, convE2_b, convE3_wm, convE3_b,
           fcE_wm, fcE_b, head_wm, head_b, fromZ_wm, fromZ_b,
           fcD_wm, fcD_b, convD1_wm, convD1_b, convD2_wm, convD2_b,
           convD3_wmT, convD3_b, x, y_onehot, eps):
    B = x.shape[0]
    Bc = 16 if B % 32 == 0 else B
    nsteps = B // Bc
    grid = (1, nsteps)

    xs = jnp.pad(x[:, 0].astype(jnp.bfloat16), ((0, 0), (1, 1), (1, 1)))
    p1 = jnp.stack([xs[:, i:i + 32:2, j:j + 32:2]
                    for i in range(4) for j in range(4)], axis=0)
    p1 = p1.reshape(16, B * 256)
    w1 = jnp.transpose(convE1_wm)
    b1 = jnp.transpose(convE1_b)

    wd2 = jnp.pad(convD2_wm.reshape(1152, 4, 64),
                  ((0, 0), (0, 0), (0, 64))).reshape(1152, 512)
    bd2 = jnp.pad(convD2_b.reshape(1, 4, 64),
                  ((0, 0), (0, 0), (0, 64))).reshape(1, 512)
    wd3 = jnp.pad(jnp.transpose(convD3_wmT).reshape(9, 64, 4),
                  ((0, 0), (0, 64), (0, 0))).reshape(1152, 4)
    bd3 = convD3_b.reshape(1, 4)

    const = lambda *shape: pl.BlockSpec(shape, lambda c, i: (0,) * len(shape))
    chunk = lambda *shape: pl.BlockSpec(
        shape, lambda c, i: (c * nsteps + i,) + (0,) * (len(shape) - 1))
    shp32 = jax.ShapeDtypeStruct((B, 32), jnp.float32)

    mu, logvar, xr = pl.pallas_call(
        _vae_kernel,
        grid=grid,
        out_shape=(shp32, shp32,
                   jax.ShapeDtypeStruct((16, B, 16, 4), jnp.float32)),
        in_specs=[
            pl.BlockSpec((16, Bc * 256), lambda c, i: (0, c * nsteps + i)),
            chunk(Bc, 32),
            chunk(Bc, 10),
            const(128, 16), const(128, 1),
            const(2048, 128), const(1, 128),
            const(2048, 256), const(1, 256),
            const(4096, 1024), const(1, 1024),
            const(1024, 64), const(1, 64),
            const(42, 1024), const(1, 1024),
            const(1024, 4096), const(1, 4096),
            const(2304, 512), const(1, 512),
            const(1152, 512), const(1, 512),
            const(1152, 4), const(1, 4),
        ],
        out_specs=(
            chunk(Bc, 32),
            chunk(Bc, 32),
            pl.BlockSpec((16, Bc, 16, 4), lambda c, i: (0, c * nsteps + i, 0, 0)),
        ),
        compiler_params=pltpu.CompilerParams(
            dimension_semantics=("parallel", "arbitrary")),
    )(p1, eps.astype(jnp.float32), y_onehot.astype(jnp.float32),
      w1, b1, convE2_wm, convE2_b, convE3_wm, convE3_b,
      fcE_wm, fcE_b, head_wm, head_b, fromZ_wm, fromZ_b, fcD_wm, fcD_b,
      convD1_wm, convD1_b, wd2, bd2, wd3, bd3)

    xr = xr.reshape(4, 4, B, 4, 4, 2, 2)
    xr = jnp.transpose(xr, (2, 3, 0, 5, 4, 1, 6)).reshape(B, 1, 32, 32)
    return mu, logvar, xr
